# baseline (speedup 1.0000x reference)
import jax
import jax.numpy as jnp
from jax.experimental import pallas as pl
from jax.experimental.pallas import tpu as pltpu

LRELU_SLOPE = 0.2
BN_EPS = 1e-5


def _lrelu(x):
    return jnp.where(x > 0, x, LRELU_SLOPE * x)


def _valid_cols(mp, c, w, wo, mc):
    row = jax.lax.broadcasted_iota(jnp.int32, (mp, c), 0)
    return (row < mc) & ((row % w) < wo)


def _valid_interior(mp, c, w, lo, hi):
    row = jax.lax.broadcasted_iota(jnp.int32, (mp, c), 0)
    h, wcol = row // w, row % w
    return (h >= lo) & (h < hi) & (wcol >= lo) & (wcol < hi)


def _affine_in(x_ref, s_ref, t_ref, w, hi):
    x = x_ref[0].astype(jnp.float32) * s_ref[...] + t_ref[...]
    x = _lrelu(x)
    mask = _valid_interior(x.shape[0], x.shape[1], w, 2, hi)
    return jnp.where(mask, x, 0.0).astype(jnp.bfloat16)


def _phases(x, hp, wp):
    c = x.shape[1]
    img = x.reshape(hp, wp, c)
    out = []
    for p in range(2):
        for q in range(2):
            ph = img[p::2, q::2, :]
            out.append(ph.reshape(ph.shape[0] * ph.shape[1], c))
    return out


def _dot(patches, w_ref, b_ref):
    y = jnp.dot(patches, w_ref[...], preferred_element_type=jnp.float32)
    return y + b_ref[...]


def _stats(y, sum_ref, sq_ref, w, wo, mc):
    ym = jnp.where(_valid_cols(y.shape[0], y.shape[1], w, wo, mc), y, 0.0)
    sum_ref[...] = jnp.sum(ym, axis=0)[None, None]
    sq_ref[...] = jnp.sum(ym * ym, axis=0)[None, None]


def _rowstore(o_ref, ym, ho, wc, wout):
    o_ref[...] = jnp.zeros_like(o_ref)
    y2 = ym[:ho * wc].reshape(ho, wc, ym.shape[1])
    for oh in range(ho):
        r0 = (2 + oh) * wout + 2
        o_ref[0, r0:r0 + wc, :] = y2[oh]


def _k2(x_ref, w_ref, b_ref, o_ref, sum_ref, sq_ref, patch_ref):
    x = x_ref[0]
    for pp in range(2):
        for qq in range(2):
            base = (2 * pp + qq) * 1225
            for kh in range(4):
                row4 = 2 * pp + kh - 4
                r4 = row4 % 4
                di = row4 // 4
                for kw in range(2):
                    vals = []
                    for kk in (kw, kw + 2):
                        col4 = 2 * qq + kk - 4
                        c4, dj = col4 % 4, col4 // 4
                        s = (4 * r4 + c4) * 1330 + (1 + di) * 35 + (1 + dj)
                        vals.append(x[s:s + 1225])
                    dst = (kh * 2 + kw) * 128
                    patch_ref[base:base + 1225, dst:dst + 128] = \
                        jnp.concatenate(vals, axis=1)
    y = _dot(patch_ref[...], w_ref, b_ref)
    m = jax.lax.broadcasted_iota(jnp.int32, (4900, 128), 0)
    bl, r = m // 1225, m % 1225
    oh = 2 * (r // 35) + (bl >> 1) - 2
    ow = 2 * (r % 35) + (bl & 1) - 2
    valid = (oh >= 0) & (oh < 65) & (ow >= 0) & (ow < 65)
    ym = jnp.where(valid, y, 0.0)
    sum_ref[...] = jnp.sum(ym, axis=0)[None, None]
    sq_ref[...] = jnp.sum(ym * ym, axis=0)[None, None]
    o_ref[0] = ym.astype(jnp.bfloat16)


def _k3(x_ref, s_ref, t_ref, w_ref, b_ref, o_ref, sum_ref, sq_ref,
        patch_ref):
    x = x_ref[0].astype(jnp.float32) * s_ref[...] + t_ref[...]
    x = _lrelu(x)
    m = jax.lax.broadcasted_iota(jnp.int32, (4900, 128), 0)
    bl, r = m // 1225, m % 1225
    oh = 2 * (r // 35) + (bl >> 1) - 2
    ow = 2 * (r % 35) + (bl & 1) - 2
    valid = (oh >= 0) & (oh < 65) & (ow >= 0) & (ow < 65)
    xb = jnp.where(valid, x, 0.0).astype(jnp.bfloat16)
    for kh in range(4):
        for kw in range(4):
            bl2 = 2 * (kh & 1) + (kw & 1)
            s = bl2 * 1225 + (kh >> 1) * 35 + (kw >> 1)
            patch_ref[:, (kh * 4 + kw) * 128:(kh * 4 + kw) * 128 + 128] = \
                xb[s:s + 1160]
    y = _dot(patch_ref[...], w_ref, b_ref)
    _stats(y, sum_ref, sq_ref, 35, 33, 1155)
    ym = jnp.where(_valid_cols(1160, 256, 35, 33, 1155), y, 0.0)
    _rowstore(o_ref, ym.astype(jnp.bfloat16), 33, 35, 38)


def _k4(x_ref, s_ref, t_ref, w_ref, b_ref, o_ref, sum_ref, sq_ref,
        patch_ref):
    xb = _affine_in(x_ref, s_ref, t_ref, 38, 35)
    for kh in range(4):
        for kw in range(4):
            off = kh * 38 + kw
            patch_ref[:, (kh * 4 + kw) * 256:(kh * 4 + kw) * 256 + 256] = \
                xb[off:off + 1296]
    y = _dot(patch_ref[...], w_ref, b_ref)
    _stats(y, sum_ref, sq_ref, 38, 34, 1292)
    ym = jnp.where(_valid_cols(1296, 512, 38, 34, 1292), y, 0.0)
    yb = ym.astype(jnp.bfloat16)
    o_ref[0, 0:78, :] = jnp.zeros((78, 512), jnp.bfloat16)
    o_ref[0, 78:78 + 1296, :] = yb
    o_ref[0, 1374:1456, :] = jnp.zeros((82, 512), jnp.bfloat16)


def _k5(x_ref, s_ref, t_ref, w_ref, b_ref, o_ref, patch_ref):
    xb = _affine_in(x_ref, s_ref, t_ref, 38, 36)
    for kh in range(4):
        for kw in range(4):
            off = kh * 38 + kw
            patch_ref[:, (kh * 4 + kw) * 512:(kh * 4 + kw) * 512 + 512] = \
                xb[off:off + 1336]
    o_ref[0] = _dot(patch_ref[...], w_ref, b_ref)


def _full(shape):
    return pl.BlockSpec((1,) + shape[1:], lambda i: (i,) + (0,) * len(shape[1:]))


def _const(shape):
    return pl.BlockSpec(shape, lambda i: (0,) * len(shape))


def _call(body, n, ins, outs, scratch=None):
    in_specs = [_full(a.shape) if a.shape[0] == n else _const(a.shape)
                for a in ins]
    out_shape = [jax.ShapeDtypeStruct(s, d) for s, d in outs]
    out_specs = [_full(s) for s, _ in outs]
    if len(outs) == 1:
        out_shape, out_specs = out_shape[0], out_specs[0]
    else:
        out_shape, out_specs = tuple(out_shape), tuple(out_specs)
    return pl.pallas_call(
        body,
        grid=(n,),
        in_specs=in_specs,
        out_specs=out_specs,
        out_shape=out_shape,
        scratch_shapes=([pltpu.VMEM(scratch, jnp.bfloat16)] if scratch
                        else []),
        compiler_params=pltpu.CompilerParams(
            dimension_semantics=("parallel",)),
    )(*ins)


def _bn_scale_shift(psum, psq, gamma, beta, count):
    s = jnp.sum(psum, axis=0)
    q = jnp.sum(psq, axis=0)
    mean = s / count
    var = jnp.maximum(q / count - mean * mean, 0.0)
    scale = gamma / jnp.sqrt(var + BN_EPS)
    shift = beta - mean * scale
    return scale, shift


def kernel(x_nchw, w0, b0, w1, b1, gamma1, beta1, w2, b2, gamma2, beta2,
           w3, b3, gamma3, beta3, w4, b4):
    n = x_nchw.shape[0]

    x = jnp.transpose(x_nchw, (0, 2, 3, 1)).astype(jnp.bfloat16)
    y0 = jax.lax.conv_general_dilated(
        x, w0[:48, :64].reshape(4, 4, 3, 64).astype(jnp.bfloat16),
        window_strides=(2, 2), padding=((2, 2), (2, 2)),
        dimension_numbers=("NHWC", "HWIO", "NHWC"),
        preferred_element_type=jnp.float32)
    y0 = _lrelu(y0 + b0[0, :64]).astype(jnp.bfloat16)
    p = jnp.pad(y0, ((0, 0), (6, 17), (6, 5), (0, 0)))
    a0 = p.reshape(n, 38, 4, 35, 4, 64).transpose(0, 2, 4, 1, 3, 5) \
          .reshape(n, 21280, 64)

    w1r = w1.reshape(4, 4, 64, 128)[:, jnp.array([0, 2, 1, 3])] \
            .reshape(1024, 128)
    y1, s1, q1 = _call(_k2, n, [a0, w1r, b1],
                       [((n, 4900, 128), jnp.bfloat16),
                        ((n, 1, 128), jnp.float32),
                        ((n, 1, 128), jnp.float32)],
                       scratch=(4900, 1024))
    sc1, sh1 = _bn_scale_shift(s1, q1, gamma1, beta1, float(n * 65 * 65))

    y2, s2, q2 = _call(_k3, n, [y1, sc1, sh1, w2, b2],
                       [((n, 1448, 256), jnp.bfloat16),
                        ((n, 1, 256), jnp.float32),
                        ((n, 1, 256), jnp.float32)],
                       scratch=(1160, 2048))
    sc2, sh2 = _bn_scale_shift(s2, q2, gamma2, beta2, float(n * 33 * 33))

    y3, s3, q3 = _call(_k4, n, [y2, sc2, sh2, w3, b3],
                       [((n, 1456, 512), jnp.bfloat16),
                        ((n, 1, 512), jnp.float32),
                        ((n, 1, 512), jnp.float32)],
                       scratch=(1296, 4096))
    sc3, sh3 = _bn_scale_shift(s3, q3, gamma3, beta3, float(n * 34 * 34))

    y4 = _call(_k5, n, [y3, sc3, sh3, w4, b4],
               [((n, 1336, 128), jnp.float32)],
               scratch=(1336, 8192))

    out = y4[:, :1330, 0].reshape(n, 35, 38)[:, :, :35]
    return out[:, None].astype(jnp.float32)


# speedup vs baseline: 1.2661x; 1.2661x over previous
import jax
import jax.numpy as jnp
from jax.experimental import pallas as pl
from jax.experimental.pallas import tpu as pltpu

LRELU_SLOPE = 0.2
BN_EPS = 1e-5


def _lrelu(x):
    return jnp.where(x > 0, x, LRELU_SLOPE * x)


def _valid_cols(mp, c, w, wo, mc):
    row = jax.lax.broadcasted_iota(jnp.int32, (mp, c), 0)
    return (row < mc) & ((row % w) < wo)


def _valid_interior(mp, c, w, lo, hi):
    row = jax.lax.broadcasted_iota(jnp.int32, (mp, c), 0)
    h, wcol = row // w, row % w
    return (h >= lo) & (h < hi) & (wcol >= lo) & (wcol < hi)


def _affine_in(x_ref, s_ref, t_ref, w, hi):
    x = x_ref[0].astype(jnp.float32) * s_ref[...] + t_ref[...]
    x = _lrelu(x)
    mask = _valid_interior(x.shape[0], x.shape[1], w, 2, hi)
    return jnp.where(mask, x, 0.0).astype(jnp.bfloat16)


def _phases(x, hp, wp):
    c = x.shape[1]
    img = x.reshape(hp, wp, c)
    out = []
    for p in range(2):
        for q in range(2):
            ph = img[p::2, q::2, :]
            out.append(ph.reshape(ph.shape[0] * ph.shape[1], c))
    return out


def _dot(patches, w_ref, b_ref):
    y = jnp.dot(patches, w_ref[...], preferred_element_type=jnp.float32)
    return y + b_ref[...]


def _stats(y, sum_ref, sq_ref, w, wo, mc):
    ym = jnp.where(_valid_cols(y.shape[0], y.shape[1], w, wo, mc), y, 0.0)
    sum_ref[...] = jnp.sum(ym, axis=0)[None, None]
    sq_ref[...] = jnp.sum(ym * ym, axis=0)[None, None]


def _rowstore(o_ref, ym, ho, wc, wout):
    o_ref[...] = jnp.zeros_like(o_ref)
    y2 = ym[:ho * wc].reshape(ho, wc, ym.shape[1])
    for oh in range(ho):
        r0 = (2 + oh) * wout + 2
        o_ref[0, r0:r0 + wc, :] = y2[oh]


def _k1(p_ref, w_ref, b_ref, o_ref):
    y = _lrelu(_dot(p_ref[0], w_ref, b_ref))
    m = jax.lax.broadcasted_iota(jnp.int32, (21280, 64), 0)
    ph, r = m // 1330, m % 1330
    oh = 4 * (r // 35) + (ph >> 2) - 6
    ow = 4 * (r % 35) + (ph & 3) - 6
    valid = (oh >= 0) & (oh < 129) & (ow >= 0) & (ow < 129)
    o_ref[0] = jnp.where(valid, y, 0.0).astype(jnp.bfloat16)


def _k2(x_ref, w_ref, b_ref, o_ref, sum_ref, sq_ref, patch_ref):
    x = x_ref[0]
    for pp in range(2):
        for qq in range(2):
            base = (2 * pp + qq) * 1225
            for kh in range(4):
                row4 = 2 * pp + kh - 4
                r4 = row4 % 4
                di = row4 // 4
                for kw in range(2):
                    vals = []
                    for kk in (kw, kw + 2):
                        col4 = 2 * qq + kk - 4
                        c4, dj = col4 % 4, col4 // 4
                        s = (4 * r4 + c4) * 1330 + (1 + di) * 35 + (1 + dj)
                        vals.append(x[s:s + 1225])
                    dst = (kh * 2 + kw) * 128
                    patch_ref[base:base + 1225, dst:dst + 128] = \
                        jnp.concatenate(vals, axis=1)
    y = _dot(patch_ref[...], w_ref, b_ref)
    m = jax.lax.broadcasted_iota(jnp.int32, (4900, 128), 0)
    bl, r = m // 1225, m % 1225
    oh = 2 * (r // 35) + (bl >> 1) - 2
    ow = 2 * (r % 35) + (bl & 1) - 2
    valid = (oh >= 0) & (oh < 65) & (ow >= 0) & (ow < 65)
    ym = jnp.where(valid, y, 0.0)
    sum_ref[...] = jnp.sum(ym, axis=0)[None, None]
    sq_ref[...] = jnp.sum(ym * ym, axis=0)[None, None]
    o_ref[0] = ym.astype(jnp.bfloat16)


def _k3(x_ref, s_ref, t_ref, w_ref, b_ref, o_ref, sum_ref, sq_ref,
        patch_ref):
    x = x_ref[0].astype(jnp.float32) * s_ref[...] + t_ref[...]
    x = _lrelu(x)
    m = jax.lax.broadcasted_iota(jnp.int32, (4900, 128), 0)
    bl, r = m // 1225, m % 1225
    oh = 2 * (r // 35) + (bl >> 1) - 2
    ow = 2 * (r % 35) + (bl & 1) - 2
    valid = (oh >= 0) & (oh < 65) & (ow >= 0) & (ow < 65)
    xb = jnp.where(valid, x, 0.0).astype(jnp.bfloat16)
    for kh in range(4):
        for kw in range(4):
            bl2 = 2 * (kh & 1) + (kw & 1)
            s = bl2 * 1225 + (kh >> 1) * 35 + (kw >> 1)
            patch_ref[:, (kh * 4 + kw) * 128:(kh * 4 + kw) * 128 + 128] = \
                xb[s:s + 1160]
    y = _dot(patch_ref[...], w_ref, b_ref)
    _stats(y, sum_ref, sq_ref, 35, 33, 1155)
    ym = jnp.where(_valid_cols(1160, 256, 35, 33, 1155), y, 0.0)
    _rowstore(o_ref, ym.astype(jnp.bfloat16), 33, 35, 38)


def _k4(x_ref, s_ref, t_ref, w_ref, b_ref, o_ref, sum_ref, sq_ref,
        patch_ref):
    xb = _affine_in(x_ref, s_ref, t_ref, 38, 35)
    for kh in range(4):
        for kw in range(4):
            off = kh * 38 + kw
            patch_ref[:, (kh * 4 + kw) * 256:(kh * 4 + kw) * 256 + 256] = \
                xb[off:off + 1296]
    y = _dot(patch_ref[...], w_ref, b_ref)
    _stats(y, sum_ref, sq_ref, 38, 34, 1292)
    ym = jnp.where(_valid_cols(1296, 512, 38, 34, 1292), y, 0.0)
    yb = ym.astype(jnp.bfloat16)
    o_ref[0, 0:78, :] = jnp.zeros((78, 512), jnp.bfloat16)
    o_ref[0, 78:78 + 1296, :] = yb
    o_ref[0, 1374:1456, :] = jnp.zeros((82, 512), jnp.bfloat16)


def _k5(x_ref, s_ref, t_ref, w_ref, b_ref, o_ref, patch_ref):
    xb = _affine_in(x_ref, s_ref, t_ref, 38, 36)
    for kh in range(4):
        for kw in range(4):
            off = kh * 38 + kw
            patch_ref[:, (kh * 4 + kw) * 512:(kh * 4 + kw) * 512 + 512] = \
                xb[off:off + 1336]
    o_ref[0] = _dot(patch_ref[...], w_ref, b_ref)


def _full(shape):
    return pl.BlockSpec((1,) + shape[1:], lambda i: (i,) + (0,) * len(shape[1:]))


def _const(shape):
    return pl.BlockSpec(shape, lambda i: (0,) * len(shape))


def _call(body, n, ins, outs, scratch=None):
    in_specs = [_full(a.shape) if a.shape[0] == n else _const(a.shape)
                for a in ins]
    out_shape = [jax.ShapeDtypeStruct(s, d) for s, d in outs]
    out_specs = [_full(s) for s, _ in outs]
    if len(outs) == 1:
        out_shape, out_specs = out_shape[0], out_specs[0]
    else:
        out_shape, out_specs = tuple(out_shape), tuple(out_specs)
    return pl.pallas_call(
        body,
        grid=(n,),
        in_specs=in_specs,
        out_specs=out_specs,
        out_shape=out_shape,
        scratch_shapes=([pltpu.VMEM(scratch, jnp.bfloat16)] if scratch
                        else []),
        compiler_params=pltpu.CompilerParams(
            dimension_semantics=("parallel",)),
    )(*ins)


def _bn_scale_shift(psum, psq, gamma, beta, count):
    s = jnp.sum(psum, axis=0)
    q = jnp.sum(psq, axis=0)
    mean = s / count
    var = jnp.maximum(q / count - mean * mean, 0.0)
    scale = gamma / jnp.sqrt(var + BN_EPS)
    shift = beta - mean * scale
    return scale, shift


def kernel(x_nchw, w0, b0, w1, b1, gamma1, beta1, w2, b2, gamma2, beta2,
           w3, b3, gamma3, beta3, w4, b4):
    n = x_nchw.shape[0]

    x = jnp.transpose(x_nchw, (0, 2, 3, 1)).astype(jnp.bfloat16)
    xp = jnp.pad(x, ((0, 0), (14, 42), (14, 18), (0, 0)))
    xt = xp.reshape(n, 39, 8, 36, 8, 3).transpose(0, 2, 4, 1, 3, 5)
    blocks = []
    for r4 in range(4):
        for c4 in range(4):
            taps = []
            for kh in range(4):
                for kw in range(4):
                    hh, ww = 2 * r4 + kh, 2 * c4 + kw
                    taps.append(xt[:, hh % 8, ww % 8,
                                   hh // 8:hh // 8 + 38,
                                   ww // 8:ww // 8 + 35, :])
            blocks.append(jnp.concatenate(taps, axis=-1)
                          .reshape(n, 1330, 48))
    p0 = jnp.concatenate(blocks, axis=1)

    a0 = _call(_k1, n, [p0, w0[:48, :64], b0[:, :64]],
               [((n, 21280, 64), jnp.bfloat16)])

    w1r = w1.reshape(4, 4, 64, 128)[:, jnp.array([0, 2, 1, 3])] \
            .reshape(1024, 128)
    y1, s1, q1 = _call(_k2, n, [a0, w1r, b1],
                       [((n, 4900, 128), jnp.bfloat16),
                        ((n, 1, 128), jnp.float32),
                        ((n, 1, 128), jnp.float32)],
                       scratch=(4900, 1024))
    sc1, sh1 = _bn_scale_shift(s1, q1, gamma1, beta1, float(n * 65 * 65))

    y2, s2, q2 = _call(_k3, n, [y1, sc1, sh1, w2, b2],
                       [((n, 1448, 256), jnp.bfloat16),
                        ((n, 1, 256), jnp.float32),
                        ((n, 1, 256), jnp.float32)],
                       scratch=(1160, 2048))
    sc2, sh2 = _bn_scale_shift(s2, q2, gamma2, beta2, float(n * 33 * 33))

    y3, s3, q3 = _call(_k4, n, [y2, sc2, sh2, w3, b3],
                       [((n, 1456, 512), jnp.bfloat16),
                        ((n, 1, 512), jnp.float32),
                        ((n, 1, 512), jnp.float32)],
                       scratch=(1296, 4096))
    sc3, sh3 = _bn_scale_shift(s3, q3, gamma3, beta3, float(n * 34 * 34))

    y4 = _call(_k5, n, [y3, sc3, sh3, w4, b4],
               [((n, 1336, 128), jnp.float32)],
               scratch=(1336, 8192))

    out = y4[:, :1330, 0].reshape(n, 35, 38)[:, :, :35]
    return out[:, None].astype(jnp.float32)
